# Initial kernel scaffold; baseline (speedup 1.0000x reference)
import jax
import jax.numpy as jnp
from jax.experimental import pallas as pl


def kernel(x, emb0, emb1, emb2, emb3, emb4, emb5, emb6, emb7, emb8, W, b):
    raise NotImplementedError("write your pallas kernel here")


# speedup vs baseline: 10.5231x; 10.5231x over previous
import functools

import jax
import jax.numpy as jnp
import numpy as np
from jax.experimental import pallas as pl

CARD = [119, 4, 12, 12, 10, 6, 6, 2, 2]
OFFS = [0, 119, 123, 135, 147, 157, 163, 169, 171]
TOT = 176
EMB = 48
HIDDEN = 256
BLOCK = 2000


def _project_body(e0, e1, e2, e3, e4, e5, e6, e7, e8, w_ref, b_ref, p_ref):
    embs = [e0, e1, e2, e3, e4, e5, e6, e7, e8]
    parts = []
    for i in range(9):
        wi = w_ref[EMB * i:EMB * (i + 1), :]
        pi = jnp.dot(embs[i][...], wi, preferred_element_type=jnp.float32,
                     precision=jax.lax.Precision.HIGHEST)
        parts.append(pi)
    parts[0] = parts[0] + b_ref[...]
    parts.append(jnp.zeros((TOT - sum(CARD), HIDDEN), jnp.float32))
    p_ref[...] = jnp.concatenate(parts, axis=0)


def _main_body(x_ref, p_ref, o_ref):
    xb = x_ref[...]
    iota = jax.lax.broadcasted_iota(jnp.int32, (BLOCK, TOT), 1)
    m = jnp.zeros((BLOCK, TOT), jnp.float32)
    for i in range(9):
        m = m + jnp.where(iota == xb[:, i:i + 1] + OFFS[i], 1.0, 0.0)
    h = jnp.dot(m, p_ref[...], preferred_element_type=jnp.float32,
                precision=jax.lax.Precision.HIGHEST)
    o_ref[...] = h * 0.5 * (1.0 + jax.lax.erf(h * np.float32(1.0 / np.sqrt(2.0))))


@functools.partial(jax.jit, static_argnames=())
def kernel(x, emb0, emb1, emb2, emb3, emb4, emb5, emb6, emb7, emb8, W, b):
    n = x.shape[0]
    p = pl.pallas_call(
        _project_body,
        out_shape=jax.ShapeDtypeStruct((TOT, HIDDEN), jnp.float32),
    )(emb0, emb1, emb2, emb3, emb4, emb5, emb6, emb7, emb8, W,
      b.reshape(1, HIDDEN))
    grid = (n // BLOCK,)
    out = pl.pallas_call(
        _main_body,
        grid=grid,
        in_specs=[
            pl.BlockSpec((BLOCK, 9), lambda i: (i, 0)),
            pl.BlockSpec((TOT, HIDDEN), lambda i: (0, 0)),
        ],
        out_specs=pl.BlockSpec((BLOCK, HIDDEN), lambda i: (i, 0)),
        out_shape=jax.ShapeDtypeStruct((n, HIDDEN), jnp.float32),
    )(x, p)
    return out


# speedup vs baseline: 25.0325x; 2.3788x over previous
import functools

import jax
import jax.numpy as jnp
import numpy as np
from jax.experimental import pallas as pl

CARD = [119, 4, 12, 12, 10, 6, 6, 2, 2]
OFFS = [0, 119, 123, 135, 147, 157, 163, 169, 171]
TOT = 176
EMB = 48
HIDDEN = 256
BLOCK = 2000


def _project_body(e0, e1, e2, e3, e4, e5, e6, e7, e8, w_ref, b_ref, p_ref):
    embs = [e0, e1, e2, e3, e4, e5, e6, e7, e8]
    parts = []
    for i in range(9):
        wi = w_ref[EMB * i:EMB * (i + 1), :]
        pi = jnp.dot(embs[i][...], wi, preferred_element_type=jnp.float32,
                     precision=jax.lax.Precision.HIGHEST)
        parts.append(pi)
    parts[0] = parts[0] + b_ref[...]
    parts.append(jnp.zeros((TOT - sum(CARD), HIDDEN), jnp.float32))
    p_ref[...] = jnp.concatenate(parts, axis=0)


def _main_body(x_ref, s_ref, t_ref, p_ref, o_ref):
    xf = x_ref[...].astype(jnp.float32)
    xg = jnp.dot(xf, s_ref[...], preferred_element_type=jnp.float32)
    m = jnp.where(xg == t_ref[...], 1.0, 0.0)
    h = jnp.dot(m, p_ref[...], preferred_element_type=jnp.float32)
    o_ref[...] = h * 0.5 * (1.0 + jax.lax.erf(h * np.float32(1.0 / np.sqrt(2.0))))


def _lane_consts():
    s = np.zeros((9, TOT), np.float32)
    t = np.full((1, TOT), -1.0, np.float32)
    for i, (off, c) in enumerate(zip(OFFS, CARD)):
        s[i, off:off + c] = 1.0
        t[0, off:off + c] = np.arange(c, dtype=np.float32)
    return jnp.asarray(s), jnp.asarray(t)


@functools.partial(jax.jit, static_argnames=())
def kernel(x, emb0, emb1, emb2, emb3, emb4, emb5, emb6, emb7, emb8, W, b):
    n = x.shape[0]
    p = pl.pallas_call(
        _project_body,
        out_shape=jax.ShapeDtypeStruct((TOT, HIDDEN), jnp.float32),
    )(emb0, emb1, emb2, emb3, emb4, emb5, emb6, emb7, emb8, W,
      b.reshape(1, HIDDEN))
    s, t = _lane_consts()
    grid = (n // BLOCK,)
    out = pl.pallas_call(
        _main_body,
        grid=grid,
        in_specs=[
            pl.BlockSpec((BLOCK, 9), lambda i: (i, 0)),
            pl.BlockSpec((9, TOT), lambda i: (0, 0)),
            pl.BlockSpec((1, TOT), lambda i: (0, 0)),
            pl.BlockSpec((TOT, HIDDEN), lambda i: (0, 0)),
        ],
        out_specs=pl.BlockSpec((BLOCK, HIDDEN), lambda i: (i, 0)),
        out_shape=jax.ShapeDtypeStruct((n, HIDDEN), jnp.float32),
    )(x, s, t, p)
    return out


# speedup vs baseline: 29.5784x; 1.1816x over previous
import functools

import jax
import jax.numpy as jnp
import numpy as np
from jax.experimental import pallas as pl

CARD = [119, 4, 12, 12, 10, 6, 6, 2, 2]
OFFS = [0, 119, 123, 135, 147, 157, 163, 169, 171]
TOT = 176
EMB = 48
HIDDEN = 256
BLOCK = 4000


def _project_body(e0, e1, e2, e3, e4, e5, e6, e7, e8, w_ref, b_ref, p_ref):
    embs = [e0, e1, e2, e3, e4, e5, e6, e7, e8]
    parts = []
    for i in range(9):
        wi = w_ref[EMB * i:EMB * (i + 1), :]
        pi = jnp.dot(embs[i][...], wi, preferred_element_type=jnp.float32,
                     precision=jax.lax.Precision.HIGHEST)
        parts.append(pi)
    parts[0] = parts[0] + b_ref[...]
    parts.append(jnp.zeros((TOT - sum(CARD), HIDDEN), jnp.float32))
    p_ref[...] = jnp.concatenate(parts, axis=0)


def _main_body(x_ref, s_ref, t_ref, p_ref, o_ref):
    xf = x_ref[...].astype(jnp.float32)
    xg = jnp.dot(xf, s_ref[...], preferred_element_type=jnp.float32)
    m = jnp.where(xg == t_ref[...], 1.0, 0.0)
    h = jnp.dot(m, p_ref[...], preferred_element_type=jnp.float32)
    o_ref[...] = h * 0.5 * (1.0 + jax.lax.erf(h * np.float32(1.0 / np.sqrt(2.0))))


def _lane_consts():
    s = np.zeros((9, TOT), np.float32)
    t = np.full((1, TOT), -1.0, np.float32)
    for i, (off, c) in enumerate(zip(OFFS, CARD)):
        s[i, off:off + c] = 1.0
        t[0, off:off + c] = np.arange(c, dtype=np.float32)
    return jnp.asarray(s), jnp.asarray(t)


@functools.partial(jax.jit, static_argnames=())
def kernel(x, emb0, emb1, emb2, emb3, emb4, emb5, emb6, emb7, emb8, W, b):
    n = x.shape[0]
    p = pl.pallas_call(
        _project_body,
        out_shape=jax.ShapeDtypeStruct((TOT, HIDDEN), jnp.float32),
    )(emb0, emb1, emb2, emb3, emb4, emb5, emb6, emb7, emb8, W,
      b.reshape(1, HIDDEN))
    s, t = _lane_consts()
    grid = (n // BLOCK,)
    out = pl.pallas_call(
        _main_body,
        grid=grid,
        in_specs=[
            pl.BlockSpec((BLOCK, 9), lambda i: (i, 0)),
            pl.BlockSpec((9, TOT), lambda i: (0, 0)),
            pl.BlockSpec((1, TOT), lambda i: (0, 0)),
            pl.BlockSpec((TOT, HIDDEN), lambda i: (0, 0)),
        ],
        out_specs=pl.BlockSpec((BLOCK, HIDDEN), lambda i: (i, 0)),
        out_shape=jax.ShapeDtypeStruct((n, HIDDEN), jnp.float32),
    )(x, s, t, p)
    return out


# speedup vs baseline: 31.9151x; 1.0790x over previous
import functools

import jax
import jax.numpy as jnp
import numpy as np
from jax.experimental import pallas as pl

CARD = [119, 4, 12, 12, 10, 6, 6, 2, 2]
OFFS = [0, 119, 123, 135, 147, 157, 163, 169, 171]
TOT = 176
EMB = 48
HIDDEN = 256
BLOCK = 10000


def _project_body(e0, e1, e2, e3, e4, e5, e6, e7, e8, w_ref, b_ref, p_ref):
    embs = [e0, e1, e2, e3, e4, e5, e6, e7, e8]
    parts = []
    for i in range(9):
        wi = w_ref[EMB * i:EMB * (i + 1), :]
        pi = jnp.dot(embs[i][...], wi, preferred_element_type=jnp.float32,
                     precision=jax.lax.Precision.HIGHEST)
        parts.append(pi)
    parts[0] = parts[0] + b_ref[...]
    parts.append(jnp.zeros((TOT - sum(CARD), HIDDEN), jnp.float32))
    p_ref[...] = jnp.concatenate(parts, axis=0)


def _main_body(x_ref, s_ref, t_ref, p_ref, o_ref):
    xf = x_ref[...].astype(jnp.float32)
    xg = jnp.dot(xf, s_ref[...], preferred_element_type=jnp.float32)
    m = jnp.where(xg == t_ref[...], 1.0, 0.0)
    h = jnp.dot(m, p_ref[...], preferred_element_type=jnp.float32)
    o_ref[...] = h * 0.5 * (1.0 + jax.lax.erf(h * np.float32(1.0 / np.sqrt(2.0))))


def _lane_consts():
    s = np.zeros((9, TOT), np.float32)
    t = np.full((1, TOT), -1.0, np.float32)
    for i, (off, c) in enumerate(zip(OFFS, CARD)):
        s[i, off:off + c] = 1.0
        t[0, off:off + c] = np.arange(c, dtype=np.float32)
    return jnp.asarray(s), jnp.asarray(t)


@functools.partial(jax.jit, static_argnames=())
def kernel(x, emb0, emb1, emb2, emb3, emb4, emb5, emb6, emb7, emb8, W, b):
    n = x.shape[0]
    p = pl.pallas_call(
        _project_body,
        out_shape=jax.ShapeDtypeStruct((TOT, HIDDEN), jnp.float32),
    )(emb0, emb1, emb2, emb3, emb4, emb5, emb6, emb7, emb8, W,
      b.reshape(1, HIDDEN))
    s, t = _lane_consts()
    grid = (n // BLOCK,)
    out = pl.pallas_call(
        _main_body,
        grid=grid,
        in_specs=[
            pl.BlockSpec((BLOCK, 9), lambda i: (i, 0)),
            pl.BlockSpec((9, TOT), lambda i: (0, 0)),
            pl.BlockSpec((1, TOT), lambda i: (0, 0)),
            pl.BlockSpec((TOT, HIDDEN), lambda i: (0, 0)),
        ],
        out_specs=pl.BlockSpec((BLOCK, HIDDEN), lambda i: (i, 0)),
        out_shape=jax.ShapeDtypeStruct((n, HIDDEN), jnp.float32),
    )(x, s, t, p)
    return out


# speedup vs baseline: 44.0536x; 1.3803x over previous
import functools

import jax
import jax.numpy as jnp
import numpy as np
from jax.experimental import pallas as pl

CARD = [119, 4, 12, 12, 10, 6, 6, 2, 2]
OFFS = [0, 119, 123, 135, 147, 157, 163, 169, 171]
TOT = 176
EMB = 48
HIDDEN = 256
BLOCK = 20000


def _project_body(e0, e1, e2, e3, e4, e5, e6, e7, e8, w_ref, b_ref, p_ref):
    embs = [e0, e1, e2, e3, e4, e5, e6, e7, e8]
    parts = []
    for i in range(9):
        wi = w_ref[EMB * i:EMB * (i + 1), :]
        pi = jnp.dot(embs[i][...], wi, preferred_element_type=jnp.float32,
                     precision=jax.lax.Precision.HIGHEST)
        parts.append(pi)
    parts[0] = parts[0] + b_ref[...]
    parts.append(jnp.zeros((TOT - sum(CARD), HIDDEN), jnp.float32))
    p_ref[...] = jnp.concatenate(parts, axis=0)


def _main_body(x_ref, s_ref, t_ref, p_ref, o_ref):
    xf = x_ref[0].astype(jnp.float32)
    xg = jax.lax.dot_general(xf, s_ref[...], (((0,), (0,)), ((), ())),
                             preferred_element_type=jnp.float32)
    m = jnp.where(xg == t_ref[...], 1.0, 0.0)
    h = jnp.dot(m, p_ref[...], preferred_element_type=jnp.float32)
    o_ref[...] = h * 0.5 * (1.0 + jax.lax.erf(h * np.float32(1.0 / np.sqrt(2.0))))


def _lane_consts():
    s = np.zeros((9, TOT), np.float32)
    t = np.full((1, TOT), -1.0, np.float32)
    for i, (off, c) in enumerate(zip(OFFS, CARD)):
        s[i, off:off + c] = 1.0
        t[0, off:off + c] = np.arange(c, dtype=np.float32)
    return jnp.asarray(s), jnp.asarray(t)


@functools.partial(jax.jit, static_argnames=())
def kernel(x, emb0, emb1, emb2, emb3, emb4, emb5, emb6, emb7, emb8, W, b):
    n = x.shape[0]
    p = pl.pallas_call(
        _project_body,
        out_shape=jax.ShapeDtypeStruct((TOT, HIDDEN), jnp.float32),
    )(emb0, emb1, emb2, emb3, emb4, emb5, emb6, emb7, emb8, W,
      b.reshape(1, HIDDEN))
    s, t = _lane_consts()
    grid = (n // BLOCK,)
    out = pl.pallas_call(
        _main_body,
        grid=grid,
        in_specs=[
            pl.BlockSpec((1, 9, BLOCK), lambda i: (i, 0, 0)),
            pl.BlockSpec((9, TOT), lambda i: (0, 0)),
            pl.BlockSpec((1, TOT), lambda i: (0, 0)),
            pl.BlockSpec((TOT, HIDDEN), lambda i: (0, 0)),
        ],
        out_specs=pl.BlockSpec((BLOCK, HIDDEN), lambda i: (i, 0)),
        out_shape=jax.ShapeDtypeStruct((n, HIDDEN), jnp.float32),
    )(x.T.reshape(9, n // BLOCK, BLOCK).transpose(1, 0, 2), s, t, p)
    return out


# speedup vs baseline: 46.7275x; 1.0607x over previous
import functools

import jax
import jax.numpy as jnp
import numpy as np
from jax.experimental import pallas as pl

CARD = [119, 4, 12, 12, 10, 6, 6, 2, 2]
OFFS = [0, 119, 123, 135, 147, 157, 163, 169, 171]
TOT = 176
EMB = 48
HIDDEN = 256
BLOCK = 25000


def _project_body(e0, e1, e2, e3, e4, e5, e6, e7, e8, w_ref, b_ref, p_ref):
    embs = [e0, e1, e2, e3, e4, e5, e6, e7, e8]
    parts = []
    for i in range(9):
        wi = w_ref[EMB * i:EMB * (i + 1), :]
        pi = jnp.dot(embs[i][...], wi, preferred_element_type=jnp.float32,
                     precision=jax.lax.Precision.HIGHEST)
        parts.append(pi)
    parts[0] = parts[0] + b_ref[...]
    parts.append(jnp.zeros((TOT - sum(CARD), HIDDEN), jnp.float32))
    p_ref[...] = jnp.concatenate(parts, axis=0)


def _main_body(x_ref, s_ref, t_ref, p_ref, o_ref):
    xf = x_ref[0].astype(jnp.float32)
    xg = jax.lax.dot_general(xf, s_ref[...], (((0,), (0,)), ((), ())),
                             preferred_element_type=jnp.float32)
    m = jnp.where(xg == t_ref[...], 1.0, 0.0)
    h = jnp.dot(m, p_ref[...], preferred_element_type=jnp.float32)
    o_ref[...] = h * 0.5 * (1.0 + jax.lax.erf(h * np.float32(1.0 / np.sqrt(2.0))))


def _lane_consts():
    s = np.zeros((9, TOT), np.float32)
    t = np.full((1, TOT), -1.0, np.float32)
    for i, (off, c) in enumerate(zip(OFFS, CARD)):
        s[i, off:off + c] = 1.0
        t[0, off:off + c] = np.arange(c, dtype=np.float32)
    return jnp.asarray(s), jnp.asarray(t)


@functools.partial(jax.jit, static_argnames=())
def kernel(x, emb0, emb1, emb2, emb3, emb4, emb5, emb6, emb7, emb8, W, b):
    n = x.shape[0]
    p = pl.pallas_call(
        _project_body,
        out_shape=jax.ShapeDtypeStruct((TOT, HIDDEN), jnp.float32),
    )(emb0, emb1, emb2, emb3, emb4, emb5, emb6, emb7, emb8, W,
      b.reshape(1, HIDDEN))
    s, t = _lane_consts()
    grid = (n // BLOCK,)
    out = pl.pallas_call(
        _main_body,
        grid=grid,
        in_specs=[
            pl.BlockSpec((1, 9, BLOCK), lambda i: (i, 0, 0)),
            pl.BlockSpec((9, TOT), lambda i: (0, 0)),
            pl.BlockSpec((1, TOT), lambda i: (0, 0)),
            pl.BlockSpec((TOT, HIDDEN), lambda i: (0, 0)),
        ],
        out_specs=pl.BlockSpec((BLOCK, HIDDEN), lambda i: (i, 0)),
        out_shape=jax.ShapeDtypeStruct((n, HIDDEN), jnp.float32),
    )(x.T.reshape(9, n // BLOCK, BLOCK).transpose(1, 0, 2), s, t, p)
    return out
